# baseline (speedup 1.0000x reference)
import functools

import jax
import jax.numpy as jnp
from jax import lax
from jax.experimental import pallas as pl
from jax.experimental.pallas import tpu as pltpu
from jax.experimental.pallas import tpu_sc as plsc

NC = 2
NS = 16
NW = NC * NS

B = 4096
L = 200
VOCAB = 1000000
D_EMB = 64
D_PAD = 128
D_HIDDEN = 512
N_ROWS = B * L
CHUNK = 128
NBUF = 4
DEPTH = 2

S = 5
ROWS_S = N_ROWS // S
ROWS_PER_W = ROWS_S // NW
NCH = ROWS_PER_W // CHUNK

BM = 4096
NBM = ROWS_S // BM


def _gather_body(idx_hbm, table_hbm, out_hbm, idx_v, *rest):
    rows = rest[:NBUF]
    gsem = rest[NBUF:2 * NBUF]
    wsem = rest[2 * NBUF:3 * NBUF]
    w = lax.axis_index("s") * NC + lax.axis_index("c")
    base = w * ROWS_PER_W

    pltpu.sync_copy(idx_hbm.at[w], idx_v)

    def gather_start(j, b):
        pltpu.async_copy(table_hbm.at[idx_v.at[j]], rows[b], gsem[b])

    def gather_wait(b):
        pltpu.make_async_copy(table_hbm.at[idx_v.at[0]], rows[b], gsem[b]).wait()

    def wb_dst(j):
        return out_hbm.at[pl.ds(base + j * CHUNK, CHUNK)]

    def wb_start(j, b):
        pltpu.async_copy(rows[b], wb_dst(j), wsem[b])

    def wb_wait(b):
        pltpu.make_async_copy(rows[b], wb_dst(0), wsem[b]).wait()

    for j in range(DEPTH):
        gather_start(j, j)

    def outer(i, carry):
        j0 = i * NBUF
        for b in range(NBUF):
            j = j0 + b
            gather_wait(b)
            wb_start(j, b)
            bn = (b + DEPTH) % NBUF
            @pl.when(j + DEPTH >= NBUF)
            def _():
                wb_wait(bn)

            @pl.when(j + DEPTH < NCH)
            def _():
                gather_start(j + DEPTH, bn)
        return carry

    lax.fori_loop(0, NCH // NBUF, outer, 0)
    for j in range(NCH - DEPTH, NCH):
        wb_wait(j % NBUF)


@functools.partial(
    pl.kernel,
    mesh=plsc.VectorSubcoreMesh(core_axis_name="c", subcore_axis_name="s"),
    out_type=jax.ShapeDtypeStruct((ROWS_S, D_PAD), jnp.float32),
    scratch_types=(
        [pltpu.VMEM((NCH, CHUNK), jnp.int32)]
        + [pltpu.VMEM((CHUNK, D_PAD), jnp.float32) for _ in range(NBUF)]
        + [pltpu.SemaphoreType.DMA for _ in range(2 * NBUF)]
    ),
)
def _sc_gather(idx_hbm, table_hbm, out_hbm, *rest):
    _gather_body(idx_hbm, table_hbm, out_hbm, *rest)


def _matmul_first(e_ref, w_ref, b_ref, o_ref):
    o_ref[...] = (
        jnp.dot(e_ref[:, :D_EMB], w_ref[...],
                preferred_element_type=jnp.float32)
        + b_ref[...]
    )


def _matmul_chained(prev_ref, e_ref, w_ref, b_ref, o_ref):
    del prev_ref
    o_ref[...] = (
        jnp.dot(e_ref[:, :D_EMB], w_ref[...],
                preferred_element_type=jnp.float32)
        + b_ref[...]
    )


def _tc_project_stage(stage, prev, emb, W, b2):
    out_shape = jax.ShapeDtypeStruct((N_ROWS, D_HIDDEN), jnp.float32)
    out_spec = pl.BlockSpec((BM, D_HIDDEN),
                            lambda i, s=stage: (s * NBM + i, 0))
    common_specs = [
        pl.BlockSpec((BM, D_PAD), lambda i: (i, 0)),
        pl.BlockSpec((D_EMB, D_HIDDEN), lambda i: (0, 0)),
        pl.BlockSpec((1, D_HIDDEN), lambda i: (0, 0)),
    ]
    if stage == 0:
        return pl.pallas_call(
            _matmul_first,
            grid=(NBM,),
            in_specs=common_specs,
            out_specs=out_spec,
            out_shape=out_shape,
        )(emb, W, b2)
    return pl.pallas_call(
        _matmul_chained,
        grid=(NBM,),
        in_specs=[pl.BlockSpec(memory_space=pltpu.MemorySpace.HBM)] + common_specs,
        out_specs=out_spec,
        out_shape=out_shape,
        input_output_aliases={0: 0},
    )(prev, emb, W, b2)


def kernel(input, table, W, b):
    idx = input.astype(jnp.int32).reshape(S, NW, NCH, CHUNK)
    b2 = b.reshape(1, D_HIDDEN)
    table128 = jnp.pad(table, ((0, 0), (0, D_PAD - D_EMB)))
    embs = [_sc_gather(idx[s], table128) for s in range(S)]
    out = None
    for s in range(S):
        out = _tc_project_stage(s, out, embs[s], W, b2)
    return out.reshape(B, L, D_HIDDEN)


# speedup vs baseline: 1.0481x; 1.0481x over previous
import functools

import jax
import jax.numpy as jnp
from jax import lax
from jax.experimental import pallas as pl
from jax.experimental.pallas import tpu as pltpu
from jax.experimental.pallas import tpu_sc as plsc

NC = 2
NS = 16
NW = NC * NS

B = 4096
L = 200
VOCAB = 1000000
D_EMB = 64
D_PAD = 128
D_HIDDEN = 512
N_ROWS = B * L
CHUNK = 128
NBUF = 4
DEPTH = 2

S = 10
ROWS_S = N_ROWS // S
ROWS_PER_W = ROWS_S // NW
NCH = ROWS_PER_W // CHUNK

BM = 4096
NBM = ROWS_S // BM


def _gather_body(idx_hbm, table_hbm, out_hbm, idx_v, *rest):
    rows = rest[:NBUF]
    gsem = rest[NBUF:2 * NBUF]
    wsem = rest[2 * NBUF:3 * NBUF]
    w = lax.axis_index("s") * NC + lax.axis_index("c")
    base = w * ROWS_PER_W

    pltpu.sync_copy(idx_hbm.at[w], idx_v)

    def gather_start(j, b):
        pltpu.async_copy(table_hbm.at[idx_v.at[j]], rows[b], gsem[b])

    def gather_wait(b):
        pltpu.make_async_copy(table_hbm.at[idx_v.at[0]], rows[b], gsem[b]).wait()

    def wb_dst(j):
        return out_hbm.at[pl.ds(base + j * CHUNK, CHUNK), pl.ds(0, D_EMB)]

    def wb_start(j, b):
        pltpu.async_copy(rows[b], wb_dst(j), wsem[b])

    def wb_wait(b):
        pltpu.make_async_copy(rows[b], wb_dst(0), wsem[b]).wait()

    for j in range(DEPTH):
        gather_start(j, j)

    def outer(i, carry):
        j0 = i * NBUF
        for b in range(NBUF):
            j = j0 + b
            gather_wait(b)
            wb_start(j, b)
            bn = (b + DEPTH) % NBUF
            @pl.when(j + DEPTH >= NBUF)
            def _():
                wb_wait(bn)

            @pl.when(j + DEPTH < NCH)
            def _():
                gather_start(j + DEPTH, bn)
        return carry

    lax.fori_loop(0, NCH // NBUF, outer, 0)
    for j in range(NCH - DEPTH, NCH):
        wb_wait(j % NBUF)


@functools.partial(
    pl.kernel,
    mesh=plsc.VectorSubcoreMesh(core_axis_name="c", subcore_axis_name="s"),
    compiler_params=pltpu.CompilerParams(use_tc_tiling_on_sc=False),
    out_type=jax.ShapeDtypeStruct((ROWS_S, D_PAD), jnp.float32),
    scratch_types=(
        [pltpu.VMEM((NCH, CHUNK), jnp.int32)]
        + [pltpu.VMEM((CHUNK, D_EMB), jnp.float32) for _ in range(NBUF)]
        + [pltpu.SemaphoreType.DMA for _ in range(2 * NBUF)]
    ),
)
def _sc_gather(idx_hbm, table_hbm, out_hbm, *rest):
    _gather_body(idx_hbm, table_hbm, out_hbm, *rest)


def _matmul_first(e_ref, w_ref, b_ref, o_ref):
    o_ref[...] = (
        jnp.dot(e_ref[:, :D_EMB], w_ref[...],
                preferred_element_type=jnp.float32)
        + b_ref[...]
    )


def _matmul_chained(prev_ref, e_ref, w_ref, b_ref, o_ref):
    del prev_ref
    o_ref[...] = (
        jnp.dot(e_ref[:, :D_EMB], w_ref[...],
                preferred_element_type=jnp.float32)
        + b_ref[...]
    )


def _tc_project_stage(stage, prev, emb, W, b2):
    out_shape = jax.ShapeDtypeStruct((N_ROWS, D_HIDDEN), jnp.float32)
    out_spec = pl.BlockSpec((BM, D_HIDDEN),
                            lambda i, s=stage: (s * NBM + i, 0))
    common_specs = [
        pl.BlockSpec((BM, D_PAD), lambda i: (i, 0)),
        pl.BlockSpec((D_EMB, D_HIDDEN), lambda i: (0, 0)),
        pl.BlockSpec((1, D_HIDDEN), lambda i: (0, 0)),
    ]
    if stage == 0:
        return pl.pallas_call(
            _matmul_first,
            grid=(NBM,),
            in_specs=common_specs,
            out_specs=out_spec,
            out_shape=out_shape,
        )(emb, W, b2)
    return pl.pallas_call(
        _matmul_chained,
        grid=(NBM,),
        in_specs=[pl.BlockSpec(memory_space=pltpu.MemorySpace.HBM)] + common_specs,
        out_specs=out_spec,
        out_shape=out_shape,
        input_output_aliases={0: 0},
    )(prev, emb, W, b2)


def kernel(input, table, W, b):
    idx = input.astype(jnp.int32).reshape(S, NW, NCH, CHUNK)
    b2 = b.reshape(1, D_HIDDEN)
    embs = [_sc_gather(idx[s], table) for s in range(S)]
    out = None
    for s in range(S):
        out = _tc_project_stage(s, out, embs[s], W, b2)
    return out.reshape(B, L, D_HIDDEN)


# speedup vs baseline: 1.0628x; 1.0141x over previous
import functools

import jax
import jax.numpy as jnp
from jax import lax
from jax.experimental import pallas as pl
from jax.experimental.pallas import tpu as pltpu
from jax.experimental.pallas import tpu_sc as plsc

NC = 2
NS = 16
NW = NC * NS

B = 4096
L = 200
VOCAB = 1000000
D_EMB = 64
D_PAD = 128
D_HIDDEN = 512
N_ROWS = B * L
CHUNK = 128
NBUF = 4
DEPTH = 2

S = 5
ROWS_S = N_ROWS // S
ROWS_PER_W = ROWS_S // NW
NCH = ROWS_PER_W // CHUNK

BM = 8192
NBM = ROWS_S // BM


def _gather_body(idx_hbm, table_hbm, out_hbm, idx_v, *rest):
    rows = rest[:NBUF]
    gsem = rest[NBUF:2 * NBUF]
    wsem = rest[2 * NBUF:3 * NBUF]
    w = lax.axis_index("s") * NC + lax.axis_index("c")
    base = w * ROWS_PER_W

    pltpu.sync_copy(idx_hbm.at[w], idx_v)

    def gather_start(j, b):
        pltpu.async_copy(table_hbm.at[idx_v.at[j]], rows[b], gsem[b])

    def gather_wait(b):
        pltpu.make_async_copy(table_hbm.at[idx_v.at[0]], rows[b], gsem[b]).wait()

    def wb_dst(j):
        return out_hbm.at[pl.ds(base + j * CHUNK, CHUNK), pl.ds(0, D_EMB)]

    def wb_start(j, b):
        pltpu.async_copy(rows[b], wb_dst(j), wsem[b])

    def wb_wait(b):
        pltpu.make_async_copy(rows[b], wb_dst(0), wsem[b]).wait()

    for j in range(DEPTH):
        gather_start(j, j)

    def outer(i, carry):
        j0 = i * NBUF
        for b in range(NBUF):
            j = j0 + b
            gather_wait(b)
            wb_start(j, b)
            bn = (b + DEPTH) % NBUF
            @pl.when(j + DEPTH >= NBUF)
            def _():
                wb_wait(bn)

            @pl.when(j + DEPTH < NCH)
            def _():
                gather_start(j + DEPTH, bn)
        return carry

    lax.fori_loop(0, NCH // NBUF, outer, 0)
    for j in range(NCH - DEPTH, NCH):
        wb_wait(j % NBUF)


@functools.partial(
    pl.kernel,
    mesh=plsc.VectorSubcoreMesh(core_axis_name="c", subcore_axis_name="s"),
    compiler_params=pltpu.CompilerParams(use_tc_tiling_on_sc=False),
    out_type=jax.ShapeDtypeStruct((ROWS_S, D_PAD), jnp.float32),
    scratch_types=(
        [pltpu.VMEM((NCH, CHUNK), jnp.int32)]
        + [pltpu.VMEM((CHUNK, D_EMB), jnp.float32) for _ in range(NBUF)]
        + [pltpu.SemaphoreType.DMA for _ in range(2 * NBUF)]
    ),
)
def _sc_gather(idx_hbm, table_hbm, out_hbm, *rest):
    _gather_body(idx_hbm, table_hbm, out_hbm, *rest)


def _matmul_first(e_ref, w_ref, b_ref, o_ref):
    o_ref[...] = (
        jnp.dot(e_ref[:, :D_EMB], w_ref[...],
                preferred_element_type=jnp.float32)
        + b_ref[...]
    )


def _matmul_chained(prev_ref, e_ref, w_ref, b_ref, o_ref):
    del prev_ref
    o_ref[...] = (
        jnp.dot(e_ref[:, :D_EMB], w_ref[...],
                preferred_element_type=jnp.float32)
        + b_ref[...]
    )


def _tc_project_stage(stage, prev, emb, W, b2):
    out_shape = jax.ShapeDtypeStruct((N_ROWS, D_HIDDEN), jnp.float32)
    out_spec = pl.BlockSpec((BM, D_HIDDEN),
                            lambda i, s=stage: (s * NBM + i, 0))
    common_specs = [
        pl.BlockSpec((BM, D_PAD), lambda i: (i, 0)),
        pl.BlockSpec((D_EMB, D_HIDDEN), lambda i: (0, 0)),
        pl.BlockSpec((1, D_HIDDEN), lambda i: (0, 0)),
    ]
    if stage == 0:
        return pl.pallas_call(
            _matmul_first,
            grid=(NBM,),
            in_specs=common_specs,
            out_specs=out_spec,
            out_shape=out_shape,
        )(emb, W, b2)
    return pl.pallas_call(
        _matmul_chained,
        grid=(NBM,),
        in_specs=[pl.BlockSpec(memory_space=pltpu.MemorySpace.HBM)] + common_specs,
        out_specs=out_spec,
        out_shape=out_shape,
        input_output_aliases={0: 0},
    )(prev, emb, W, b2)


def kernel(input, table, W, b):
    idx = input.astype(jnp.int32).reshape(S, NW, NCH, CHUNK)
    b2 = b.reshape(1, D_HIDDEN)
    embs = [_sc_gather(idx[s], table) for s in range(S)]
    out = None
    for s in range(S):
        out = _tc_project_stage(s, out, embs[s], W, b2)
    return out.reshape(B, L, D_HIDDEN)
